# Initial kernel scaffold; baseline (speedup 1.0000x reference)
import jax
import jax.numpy as jnp
from jax.experimental import pallas as pl


def kernel(pred, y_true):
    raise NotImplementedError("write your pallas kernel here")


# speedup vs baseline: 1.0215x; 1.0215x over previous
import functools

import jax
import jax.numpy as jnp
from jax.experimental import pallas as pl
from jax.experimental.pallas import tpu as pltpu

MARGIN_ = 0.02
N_ = 4096
BR_ = 512


def _loss_kernel(p_col, y_col, p_row, y_row, num_ref, den_ref):
    i = pl.program_id(0)

    @pl.when(i == 0)
    def _init():
        num_ref[0, 0] = 0.0
        den_ref[0, 0] = 0.0

    pc = p_col[:, :]
    yc = y_col[:, :]
    pr = p_row[:, :]
    yr = y_row[:, :]

    hinge = jnp.maximum((MARGIN_ - pc) + pr, 0.0)
    weight = jnp.maximum(yc - yr, 0.0)

    num_ref[0, 0] += jnp.sum(hinge * weight)
    den_ref[0, 0] += jnp.sum(weight)


@jax.jit
def kernel(pred, y_true):
    p = pred.reshape(-1).astype(jnp.float32)
    y = y_true.reshape(-1).astype(jnp.float32)
    n = p.shape[0]

    grid = (n // BR_,)
    num, den = pl.pallas_call(
        _loss_kernel,
        grid=grid,
        in_specs=[
            pl.BlockSpec((BR_, 1), lambda i: (i, 0)),
            pl.BlockSpec((BR_, 1), lambda i: (i, 0)),
            pl.BlockSpec((1, n), lambda i: (0, 0)),
            pl.BlockSpec((1, n), lambda i: (0, 0)),
        ],
        out_specs=[
            pl.BlockSpec(memory_space=pltpu.SMEM),
            pl.BlockSpec(memory_space=pltpu.SMEM),
        ],
        out_shape=[
            jax.ShapeDtypeStruct((1, 1), jnp.float32),
            jax.ShapeDtypeStruct((1, 1), jnp.float32),
        ],
    )(p.reshape(n, 1), y.reshape(n, 1), p.reshape(1, n), y.reshape(1, n))

    return num[0, 0] / (den[0, 0] + 1e-8)
